# baseline (speedup 1.0000x reference)
import functools

import jax
import jax.numpy as jnp
from jax import lax
from jax.experimental import pallas as pl
from jax.experimental.pallas import tpu as pltpu
from jax.experimental.pallas import tpu_sc as plsc

N = 10000
E = 320000
C = 4
D = 128
CD = C * D
NPAD = 10240
RPT = NPAD // 16
EPT = E // 32
NB = 10
BN = 1024
CAP = 1280
G = 16
NBUF = 4
_ZK = 25


@functools.cache
def _build_deg_kernel():
    mesh = plsc.VectorSubcoreMesh(
        core_axis_name="c", subcore_axis_name="s",
        num_cores=2, num_subcores=16)
    return functools.partial(
        pl.kernel,
        out_type=(
            jax.ShapeDtypeStruct((2, NPAD), jnp.float32),
            jax.ShapeDtypeStruct((32, NB, CAP), jnp.int32),
            jax.ShapeDtypeStruct((32, NB, CAP), jnp.int32),
            jax.ShapeDtypeStruct((32, NB, CAP), jnp.float32),
            jax.ShapeDtypeStruct((32, 16), jnp.int32),
        ),
        mesh=mesh,
        scratch_types=[
            pltpu.VMEM((EPT,), jnp.int32),
            pltpu.VMEM((EPT,), jnp.int32),
            pltpu.VMEM((EPT,), jnp.float32),
            pltpu.VMEM((RPT,), jnp.float32),
            pltpu.VMEM((NB, CAP), jnp.int32),
            pltpu.VMEM((NB, CAP), jnp.int32),
            pltpu.VMEM((NB, CAP), jnp.float32),
            pltpu.VMEM((16,), jnp.int32),
            pltpu.VMEM_SHARED((NPAD,), jnp.float32),
            pltpu.SemaphoreType.DMA,
        ],
        compiler_params=pltpu.CompilerParams(
            needs_layout_passes=False, use_tc_tiling_on_sc=False),
    )(_deg_body)


def _deg_body(src_hbm, dst_hbm, w_hbm,
              degp_hbm, psrc_hbm, pdst_hbm, pw_hbm, cnts_hbm,
              src_v, dst_v, w_v, zbuf, ps_v, pd_v, pw_v, cnt_v, acc, sem):
    ci = lax.axis_index("c")
    si = lax.axis_index("s")
    t = ci * 16 + si
    base = t * EPT
    pltpu.sync_copy(src_hbm.at[pl.ds(base, EPT)], src_v)
    pltpu.sync_copy(dst_hbm.at[pl.ds(base, EPT)], dst_v)
    pltpu.sync_copy(w_hbm.at[pl.ds(base, EPT)], w_v)
    zeros = jnp.zeros((G,), jnp.float32)
    izeros = jnp.zeros((G,), jnp.int32)

    def _zero(i, carry):
        zbuf[pl.ds(i * G, G)] = zeros
        return carry

    lax.fori_loop(0, RPT // G, _zero, 0)
    pltpu.sync_copy(zbuf, acc.at[pl.ds(si * RPT, RPT)])
    plsc.subcore_barrier()

    def _super(h, carry):
        for b in range(_ZK):
            off = (h * _ZK + b) * G
            d16 = dst_v[pl.ds(off, G)]
            pltpu.async_copy(w_v.at[pl.ds(off, G)], acc.at[d16], sem,
                             add=True)
        for b in range(_ZK):
            off = (h * _ZK + b) * G
            d16 = dst_v[pl.ds(off, G)]
            pltpu.make_async_copy(w_v.at[pl.ds(off, G)], acc.at[d16],
                                  sem).wait()
        return carry

    lax.fori_loop(0, (EPT // G) // _ZK, _super, 0)

    def _prefill(i, carry):
        for k in range(NB):
            ps_v[k, pl.ds(i * G, G)] = izeros
            pd_v[k, pl.ds(i * G, G)] = izeros + k * BN
            pw_v[k, pl.ds(i * G, G)] = zeros
        return carry

    lax.fori_loop(0, CAP // G, _prefill, 0)

    lane = lax.broadcasted_iota(jnp.int32, (G,), 0)
    cvec = izeros
    for k in range(NB):
        def _scan(g, cur):
            sl = pl.ds(g * G, G)
            d16 = dst_v[sl]
            m = lax.shift_right_logical(d16, 10) == k
            at = jnp.minimum(cur, CAP - G)
            plsc.store_compressed(ps_v.at[k, pl.ds(at, G)], src_v[sl],
                                  mask=m)
            plsc.store_compressed(pd_v.at[k, pl.ds(at, G)], d16, mask=m)
            plsc.store_compressed(pw_v.at[k, pl.ds(at, G)], w_v[sl],
                                  mask=m)
            return cur + plsc.all_reduce_population_count(m)[0]

        cur_k = lax.fori_loop(0, EPT // G, _scan, jnp.int32(0))
        cvec = cvec + jnp.where(lane == k, cur_k, 0)
    cnt_v[...] = cvec

    pltpu.sync_copy(ps_v, psrc_hbm.at[t])
    pltpu.sync_copy(pd_v, pdst_hbm.at[t])
    pltpu.sync_copy(pw_v, pw_hbm.at[t])
    pltpu.sync_copy(cnt_v, cnts_hbm.at[t])

    plsc.subcore_barrier()
    pltpu.sync_copy(acc.at[pl.ds(si * RPT, RPT)],
                    degp_hbm.at[ci, pl.ds(si * RPT, RPT)])


def _prep_body(degs_ref, dis_ref):
    deg = degs_ref[:, 0:1] + degs_ref[:, 1:2] + 1.0
    dis_ref[...] = lax.rsqrt(deg)


_BN_B = 1024
_prep_kernel = pl.pallas_call(
    _prep_body,
    grid=(NPAD // _BN_B,),
    in_specs=[pl.BlockSpec((_BN_B, 2), lambda i: (i, 0))],
    out_specs=[pl.BlockSpec((_BN_B, 1), lambda i: (i, 0))],
    out_shape=[jax.ShapeDtypeStruct((NPAD, 1), jnp.float32)],
)


@functools.cache
def _build_msg_kernel():
    mesh = plsc.VectorSubcoreMesh(
        core_axis_name="c", subcore_axis_name="s",
        num_cores=2, num_subcores=16)
    return functools.partial(
        pl.kernel,
        out_type=jax.ShapeDtypeStruct((NPAD, CD), jnp.float32),
        mesh=mesh,
        scratch_types=[
            pltpu.VMEM((2 * CAP,), jnp.int32),
            pltpu.VMEM((2 * CAP,), jnp.int32),
            pltpu.VMEM((2 * CAP,), jnp.float32),
            pltpu.VMEM((NPAD + G,), jnp.float32),
            pltpu.VMEM((BN // 16, CD), jnp.float32),
            pltpu.VMEM((NBUF, G, CD), jnp.float32),
            pltpu.VMEM((G,), jnp.float32),
            pltpu.VMEM((2, 16), jnp.int32),
            pltpu.VMEM_SHARED((BN, CD), jnp.float32),
            [pltpu.SemaphoreType.DMA] * NBUF,
            [pltpu.SemaphoreType.DMA] * NBUF,
        ],
        compiler_params=pltpu.CompilerParams(
            needs_layout_passes=False, use_tc_tiling_on_sc=False),
    )(_msg_body)


def _msg_body(x_hbm, dis_hbm, psrc_hbm, pdst_hbm, pw_hbm,
              cnts_hbm, agg_hbm, src_v, dst_v, w_v, dis_v, xibuf, rowbuf,
              nbuf, cnt_v, acc, gsems, ssems):
    ci = lax.axis_index("c")
    si = lax.axis_index("s")
    pltpu.sync_copy(dis_hbm, dis_v.at[pl.ds(0, NPAD)])
    lane = lax.broadcasted_iota(jnp.int32, (G,), 0)
    nrows = BN // 16

    def _pass(p, carry):
        k = ci * 5 + p
        nbase = k * BN
        for j in range(2):
            t = 2 * si + j
            sl = pl.ds(j * CAP, CAP)
            pltpu.sync_copy(psrc_hbm.at[t, k], src_v.at[sl])
            pltpu.sync_copy(pdst_hbm.at[t, k], dst_v.at[sl])
            pltpu.sync_copy(pw_hbm.at[t, k], w_v.at[sl])
            pltpu.sync_copy(cnts_hbm.at[t], cnt_v.at[j])
        n0 = nbase + si * nrows

        def _iload(rg, c4):
            pltpu.sync_copy(x_hbm.at[pl.ds(n0 + rg * G, G)],
                            xibuf.at[pl.ds(rg * G, G)])
            return c4

        lax.fori_loop(0, nrows // G, _iload, 0)

        def _iscale(r, c4):
            qv = dis_v[pl.ds(n0 + r, G)]
            sc = qv[0] * qv[0]
            for jj in range(CD // G):
                xibuf[r, pl.ds(jj * G, G)] = xibuf[r, pl.ds(jj * G, G)] * sc
            return c4

        lax.fori_loop(0, nrows, _iscale, 0)

        def _iout(rg, c4):
            pltpu.sync_copy(xibuf.at[pl.ds(rg * G, G)],
                            acc.at[pl.ds(si * nrows + rg * G, G)])
            return c4

        lax.fori_loop(0, nrows // G, _iout, 0)
        plsc.subcore_barrier()

        for j in range(2):
            jbase = j * CAP
            cnt = jnp.sum(jnp.where(lane == k, cnt_v[j], 0))
            ng = (cnt + (G - 1)) >> 4

            def _gat(g, b):
                i16 = src_v[pl.ds(jbase + g * G, G)]
                pltpu.async_copy(x_hbm.at[i16], rowbuf.at[b], gsems[b])

            def _gat_desc(g, b):
                i16 = src_v[pl.ds(jbase + g * G, G)]
                return pltpu.make_async_copy(x_hbm.at[i16], rowbuf.at[b],
                                             gsems[b])

            def _scat_desc(g, b):
                d16 = dst_v[pl.ds(jbase + g * G, G)] - nbase
                return pltpu.make_async_copy(rowbuf.at[b], acc.at[d16],
                                             ssems[b])

            for b in range(NBUF):
                @pl.when(b < ng)
                def _pro():
                    _gat(b, b)

            def _outer(h, carry2):
                for b in range(NBUF):
                    g = h * NBUF + b

                    @pl.when(g < ng)
                    def _work():
                        off = pl.ds(jbase + g * G, G)
                        s16 = src_v[off]
                        d16 = dst_v[off]
                        w16 = w_v[off]
                        n16 = (plsc.load_gather(dis_v, [s16]) * w16
                               * plsc.load_gather(dis_v, [d16]))
                        _gat_desc(g, b).wait()

                        def _scale(r, c3):
                            sc = jnp.take(n16, lane * 0 + r)
                            for jj in range(CD // G):
                                rowbuf[b, r, pl.ds(jj * G, G)] = (
                                    rowbuf[b, r, pl.ds(jj * G, G)] * sc)
                            return c3

                        lax.fori_loop(0, G, _scale, 0)
                        d16b = dst_v[off] - nbase
                        pltpu.async_copy(rowbuf.at[b], acc.at[d16b],
                                         ssems[b], add=True)
                for b in range(NBUF):
                    g2 = h * NBUF + b + NBUF

                    @pl.when(g2 < ng)
                    def _refill():
                        _scat_desc(g2 - NBUF, b).wait()
                        _gat(g2, b)
                return carry2

            lax.fori_loop(0, (ng + NBUF - 1) // NBUF, _outer, 0)
            for b in range(NBUF):
                gb = ((ng - 1 - b) // NBUF) * NBUF + b

                @pl.when(ng >= b + 1)
                def _drain():
                    _scat_desc(gb, b).wait()

        plsc.subcore_barrier()
        pltpu.sync_copy(
            acc.at[pl.ds(si * nrows, nrows)],
            agg_hbm.at[pl.ds(nbase + si * nrows, nrows)])
        plsc.subcore_barrier()
        return carry

    lax.fori_loop(0, NB // 2, _pass, 0)


def _mm_body(agg_ref, w_ref, b_ref, out_ref):
    for c in range(C):
        cs = pl.ds(c * D, D)
        out_ref[:, cs] = (jnp.dot(agg_ref[:, cs], w_ref[cs, :],
                                  preferred_element_type=jnp.float32)
                          + b_ref[:, cs])


_BN_D = 1024
_mm_kernel = pl.pallas_call(
    _mm_body,
    grid=(NPAD // _BN_D,),
    in_specs=[pl.BlockSpec((_BN_D, CD), lambda i: (i, 0)),
              pl.BlockSpec((CD, D), lambda i: (0, 0)),
              pl.BlockSpec((1, CD), lambda i: (0, 0))],
    out_specs=pl.BlockSpec((_BN_D, CD), lambda i: (i, 0)),
    out_shape=jax.ShapeDtypeStruct((NPAD, CD), jnp.float32),
)


def kernel(x, edge_index, edge_weight, W, b):
    src = edge_index[0]
    dst = edge_index[1]
    x2 = jnp.pad(x.reshape(N, CD), ((0, NPAD - N), (0, 0)))
    degp, psrc, pdst, pw, cnts = _build_deg_kernel()(src, dst, edge_weight)
    dis2, = _prep_kernel(degp.T)
    agg = _build_msg_kernel()(x2, dis2.reshape(NPAD),
                              psrc, pdst, pw, cnts)
    out2 = _mm_kernel(agg, W.reshape(CD, D), b.reshape(1, CD))
    return out2.reshape(NPAD, C, D)[:N]


# speedup vs baseline: 1.0327x; 1.0327x over previous
import functools

import jax
import jax.numpy as jnp
from jax import lax
from jax.experimental import pallas as pl
from jax.experimental.pallas import tpu as pltpu
from jax.experimental.pallas import tpu_sc as plsc

N = 10000
E = 320000
C = 4
D = 128
CD = C * D
NPAD = 10240
RPT = NPAD // 16
EPT = E // 32
NB = 10
BN = 1024
CAP = 1280
G = 16
NBUF = 8
_ZK = 25


@functools.cache
def _build_deg_kernel():
    mesh = plsc.VectorSubcoreMesh(
        core_axis_name="c", subcore_axis_name="s",
        num_cores=2, num_subcores=16)
    return functools.partial(
        pl.kernel,
        out_type=(
            jax.ShapeDtypeStruct((2, NPAD), jnp.float32),
            jax.ShapeDtypeStruct((32, NB, CAP), jnp.int32),
            jax.ShapeDtypeStruct((32, NB, CAP), jnp.int32),
            jax.ShapeDtypeStruct((32, NB, CAP), jnp.float32),
            jax.ShapeDtypeStruct((32, 16), jnp.int32),
        ),
        mesh=mesh,
        scratch_types=[
            pltpu.VMEM((EPT,), jnp.int32),
            pltpu.VMEM((EPT,), jnp.int32),
            pltpu.VMEM((EPT,), jnp.float32),
            pltpu.VMEM((RPT,), jnp.float32),
            pltpu.VMEM((NB, CAP), jnp.int32),
            pltpu.VMEM((NB, CAP), jnp.int32),
            pltpu.VMEM((NB, CAP), jnp.float32),
            pltpu.VMEM((16,), jnp.int32),
            pltpu.VMEM_SHARED((NPAD,), jnp.float32),
            pltpu.SemaphoreType.DMA,
        ],
        compiler_params=pltpu.CompilerParams(
            needs_layout_passes=False, use_tc_tiling_on_sc=False),
    )(_deg_body)


def _deg_body(src_hbm, dst_hbm, w_hbm,
              degp_hbm, psrc_hbm, pdst_hbm, pw_hbm, cnts_hbm,
              src_v, dst_v, w_v, zbuf, ps_v, pd_v, pw_v, cnt_v, acc, sem):
    ci = lax.axis_index("c")
    si = lax.axis_index("s")
    t = ci * 16 + si
    base = t * EPT
    pltpu.sync_copy(src_hbm.at[pl.ds(base, EPT)], src_v)
    pltpu.sync_copy(dst_hbm.at[pl.ds(base, EPT)], dst_v)
    pltpu.sync_copy(w_hbm.at[pl.ds(base, EPT)], w_v)
    zeros = jnp.zeros((G,), jnp.float32)
    izeros = jnp.zeros((G,), jnp.int32)

    def _zero(i, carry):
        zbuf[pl.ds(i * G, G)] = zeros
        return carry

    lax.fori_loop(0, RPT // G, _zero, 0)
    pltpu.sync_copy(zbuf, acc.at[pl.ds(si * RPT, RPT)])
    plsc.subcore_barrier()

    def _super(h, carry):
        for b in range(_ZK):
            off = (h * _ZK + b) * G
            d16 = dst_v[pl.ds(off, G)]
            pltpu.async_copy(w_v.at[pl.ds(off, G)], acc.at[d16], sem,
                             add=True)
        for b in range(_ZK):
            off = (h * _ZK + b) * G
            d16 = dst_v[pl.ds(off, G)]
            pltpu.make_async_copy(w_v.at[pl.ds(off, G)], acc.at[d16],
                                  sem).wait()
        return carry

    lax.fori_loop(0, (EPT // G) // _ZK, _super, 0)

    def _prefill(i, carry):
        for k in range(NB):
            ps_v[k, pl.ds(i * G, G)] = izeros
            pd_v[k, pl.ds(i * G, G)] = izeros + k * BN
            pw_v[k, pl.ds(i * G, G)] = zeros
        return carry

    lax.fori_loop(0, CAP // G, _prefill, 0)

    lane = lax.broadcasted_iota(jnp.int32, (G,), 0)
    cvec = izeros
    for k in range(NB):
        def _scan(g, cur):
            sl = pl.ds(g * G, G)
            d16 = dst_v[sl]
            m = lax.shift_right_logical(d16, 10) == k
            at = jnp.minimum(cur, CAP - G)
            plsc.store_compressed(ps_v.at[k, pl.ds(at, G)], src_v[sl],
                                  mask=m)
            plsc.store_compressed(pd_v.at[k, pl.ds(at, G)], d16, mask=m)
            plsc.store_compressed(pw_v.at[k, pl.ds(at, G)], w_v[sl],
                                  mask=m)
            return cur + plsc.all_reduce_population_count(m)[0]

        cur_k = lax.fori_loop(0, EPT // G, _scan, jnp.int32(0))
        cvec = cvec + jnp.where(lane == k, cur_k, 0)
    cnt_v[...] = cvec

    pltpu.sync_copy(ps_v, psrc_hbm.at[t])
    pltpu.sync_copy(pd_v, pdst_hbm.at[t])
    pltpu.sync_copy(pw_v, pw_hbm.at[t])
    pltpu.sync_copy(cnt_v, cnts_hbm.at[t])

    plsc.subcore_barrier()
    pltpu.sync_copy(acc.at[pl.ds(si * RPT, RPT)],
                    degp_hbm.at[ci, pl.ds(si * RPT, RPT)])


def _prep_body(degs_ref, x_ref, dis_ref, init_ref):
    deg = degs_ref[:, 0:1] + degs_ref[:, 1:2] + 1.0
    dis = lax.rsqrt(deg)
    dis_ref[...] = dis
    init_ref[...] = x_ref[...] * (dis * dis)


_BN_B = 1024
_prep_kernel = pl.pallas_call(
    _prep_body,
    grid=(NPAD // _BN_B,),
    in_specs=[pl.BlockSpec((_BN_B, 2), lambda i: (i, 0)),
              pl.BlockSpec((_BN_B, CD), lambda i: (i, 0))],
    out_specs=[pl.BlockSpec((_BN_B, 1), lambda i: (i, 0)),
               pl.BlockSpec((_BN_B, CD), lambda i: (i, 0))],
    out_shape=[jax.ShapeDtypeStruct((NPAD, 1), jnp.float32),
               jax.ShapeDtypeStruct((NPAD, CD), jnp.float32)],
)


@functools.cache
def _build_msg_kernel():
    mesh = plsc.VectorSubcoreMesh(
        core_axis_name="c", subcore_axis_name="s",
        num_cores=2, num_subcores=16)
    return functools.partial(
        pl.kernel,
        out_type=jax.ShapeDtypeStruct((NPAD, CD), jnp.float32),
        mesh=mesh,
        scratch_types=[
            pltpu.VMEM((2 * CAP,), jnp.int32),
            pltpu.VMEM((2 * CAP,), jnp.int32),
            pltpu.VMEM((2 * CAP,), jnp.float32),
            pltpu.VMEM((NPAD,), jnp.float32),
            pltpu.VMEM((NBUF, G, CD), jnp.float32),
            pltpu.VMEM((G,), jnp.float32),
            pltpu.VMEM((2, 16), jnp.int32),
            pltpu.VMEM_SHARED((BN, CD), jnp.float32),
            [pltpu.SemaphoreType.DMA] * NBUF,
            [pltpu.SemaphoreType.DMA] * NBUF,
        ],
        compiler_params=pltpu.CompilerParams(
            needs_layout_passes=False, use_tc_tiling_on_sc=False),
    )(_msg_body)


def _msg_body(x_hbm, init_hbm, dis_hbm, psrc_hbm, pdst_hbm, pw_hbm,
              cnts_hbm, agg_hbm, src_v, dst_v, w_v, dis_v, rowbuf,
              nbuf, cnt_v, acc, gsems, ssems):
    ci = lax.axis_index("c")
    si = lax.axis_index("s")
    pltpu.sync_copy(dis_hbm, dis_v)
    lane = lax.broadcasted_iota(jnp.int32, (G,), 0)
    nrows = BN // 16

    def _pass(p, carry):
        k = ci * 5 + p
        nbase = k * BN
        for j in range(2):
            t = 2 * si + j
            sl = pl.ds(j * CAP, CAP)
            pltpu.sync_copy(psrc_hbm.at[t, k], src_v.at[sl])
            pltpu.sync_copy(pdst_hbm.at[t, k], dst_v.at[sl])
            pltpu.sync_copy(pw_hbm.at[t, k], w_v.at[sl])
            pltpu.sync_copy(cnts_hbm.at[t], cnt_v.at[j])
        pltpu.sync_copy(
            init_hbm.at[pl.ds(nbase + si * nrows, nrows)],
            acc.at[pl.ds(si * nrows, nrows)])
        plsc.subcore_barrier()

        for j in range(2):
            jbase = j * CAP
            cnt = jnp.sum(jnp.where(lane == k, cnt_v[j], 0))
            ng = (cnt + (G - 1)) >> 4

            def _gat(g, b):
                i16 = src_v[pl.ds(jbase + g * G, G)]
                pltpu.async_copy(x_hbm.at[i16], rowbuf.at[b], gsems[b])

            def _gat_desc(g, b):
                i16 = src_v[pl.ds(jbase + g * G, G)]
                return pltpu.make_async_copy(x_hbm.at[i16], rowbuf.at[b],
                                             gsems[b])

            def _scat_desc(g, b):
                d16 = dst_v[pl.ds(jbase + g * G, G)] - nbase
                return pltpu.make_async_copy(rowbuf.at[b], acc.at[d16],
                                             ssems[b])

            for b in range(NBUF):
                @pl.when(b < ng)
                def _pro():
                    _gat(b, b)

            def _outer(h, carry2):
                for b in range(NBUF):
                    g = h * NBUF + b

                    @pl.when(g < ng)
                    def _work():
                        off = pl.ds(jbase + g * G, G)
                        s16 = src_v[off]
                        d16 = dst_v[off]
                        w16 = w_v[off]
                        n16 = (plsc.load_gather(dis_v, [s16]) * w16
                               * plsc.load_gather(dis_v, [d16]))
                        _gat_desc(g, b).wait()

                        def _scale(r, c3):
                            sc = jnp.take(n16, lane * 0 + r)
                            for jj in range(CD // G):
                                rowbuf[b, r, pl.ds(jj * G, G)] = (
                                    rowbuf[b, r, pl.ds(jj * G, G)] * sc)
                            return c3

                        lax.fori_loop(0, G, _scale, 0)
                        d16b = dst_v[off] - nbase
                        pltpu.async_copy(rowbuf.at[b], acc.at[d16b],
                                         ssems[b], add=True)
                for b in range(NBUF):
                    g2 = h * NBUF + b + NBUF

                    @pl.when(g2 < ng)
                    def _refill():
                        _scat_desc(g2 - NBUF, b).wait()
                        _gat(g2, b)
                return carry2

            lax.fori_loop(0, (ng + NBUF - 1) // NBUF, _outer, 0)
            for b in range(NBUF):
                gb = ((ng - 1 - b) // NBUF) * NBUF + b

                @pl.when(ng >= b + 1)
                def _drain():
                    _scat_desc(gb, b).wait()

        plsc.subcore_barrier()
        pltpu.sync_copy(
            acc.at[pl.ds(si * nrows, nrows)],
            agg_hbm.at[pl.ds(nbase + si * nrows, nrows)])
        plsc.subcore_barrier()
        return carry

    lax.fori_loop(0, NB // 2, _pass, 0)


def _mm_body(agg_ref, w_ref, b_ref, out_ref):
    for c in range(C):
        cs = pl.ds(c * D, D)
        out_ref[:, cs] = (jnp.dot(agg_ref[:, cs], w_ref[cs, :],
                                  preferred_element_type=jnp.float32)
                          + b_ref[:, cs])


_BN_D = 1024
_mm_kernel = pl.pallas_call(
    _mm_body,
    grid=(NPAD // _BN_D,),
    in_specs=[pl.BlockSpec((_BN_D, CD), lambda i: (i, 0)),
              pl.BlockSpec((CD, D), lambda i: (0, 0)),
              pl.BlockSpec((1, CD), lambda i: (0, 0))],
    out_specs=pl.BlockSpec((_BN_D, CD), lambda i: (i, 0)),
    out_shape=jax.ShapeDtypeStruct((NPAD, CD), jnp.float32),
)


def kernel(x, edge_index, edge_weight, W, b):
    src = edge_index[0]
    dst = edge_index[1]
    x2 = jnp.pad(x.reshape(N, CD), ((0, NPAD - N), (0, 0)))
    degp, psrc, pdst, pw, cnts = _build_deg_kernel()(src, dst, edge_weight)
    dis2, init = _prep_kernel(degp.T, x2)
    agg = _build_msg_kernel()(x2, init, dis2.reshape(NPAD),
                              psrc, pdst, pw, cnts)
    out2 = _mm_kernel(agg, W.reshape(CD, D), b.reshape(1, CD))
    return out2.reshape(NPAD, C, D)[:N]
